# baseline (speedup 1.0000x reference)
import jax
import jax.numpy as jnp
from jax import lax
from jax.experimental import pallas as pl
from jax.experimental.pallas import tpu as pltpu
from jax.experimental.pallas import tpu_sc as plsc

K_RATIO_ = 0.5
BLK = 2000
LANES = 16
CHUNK = 4096
NPAD = LANES * CHUNK
NREAL = 50000
WIN = 128
NWIN = CHUNK // WIN
RADIX = 256
NOUT = 25600
KEYBASE = 32768


def _logits_body(h_ref, w_ref, out_ref):
    z = h_ref[...]
    w = w_ref[...]
    lg = jax.lax.dot_general(
        z.astype(jnp.bfloat16), w.astype(jnp.bfloat16),
        dimension_numbers=(((2,), (1,)), ((), ())),
        preferred_element_type=jnp.float32,
    )
    out_ref[...] = lg[None]


def _logits(h, W):
    B, N, D = h.shape
    grid = (N // BLK,)
    out = pl.pallas_call(
        _logits_body,
        grid=grid,
        in_specs=[
            pl.BlockSpec((B, BLK, D), lambda j: (0, j, 0)),
            pl.BlockSpec((1, D), lambda j: (0, 0)),
        ],
        out_specs=pl.BlockSpec((1, B, BLK, 1), lambda j: (j, 0, 0, 0)),
        out_shape=jax.ShapeDtypeStruct((N // BLK, B, BLK, 1), jnp.float32),
    )(h, W)
    return out[..., 0].transpose(1, 0, 2).reshape(B, N)


def _sort_body(scores_hbm, oidx_hbm, okey_hbm, sphbm,
               win_f, win_i, ukT, hist, cnt, dst):
    c = lax.axis_index("c")
    s = lax.axis_index("s")
    b = s * 2 + c
    lidx = lax.iota(jnp.int32, LANES)
    zeros16 = jnp.zeros((LANES,), jnp.int32)
    ones = jnp.ones((LANES,), jnp.int32)

    @pl.when(b < 8)
    def _run():
        def ldw(w, _):
            pltpu.sync_copy(scores_hbm.at[b].at[pl.ds(w * 2048, 2048)], win_f)

            def xf(j, _):
                x = win_f[pl.ds(j * 16, 16)]
                ukT[pl.ds(w * 2048 + j * 16, 16)] = (
                    lax.bitcast_convert_type(x, jnp.int32) ^ -1)
                return 0
            lax.fori_loop(0, 128, xf, 0)
            return 0
        lax.fori_loop(0, NREAL // 2048 + 1, ldw, 0)

        for p in range(4):
            shift = 8 * p

            def zero(d, _):
                hist[pl.ds(d * 16, 16)] = zeros16
                return 0
            lax.fori_loop(0, RADIX * LANES // 16, zero, 0)

            def sweep(w, t, permute):
                o = w * WIN + t
                if p == 0:
                    iv = lidx * CHUNK + o
                else:
                    iv = win_i[pl.ds(t * 16, 16)]
                m = (lidx * CHUNK + o) < NREAL
                kv = plsc.load_gather(ukT, [iv], mask=m)
                d = lax.shift_right_logical(kv, shift) & 255
                addr = d * LANES + lidx
                if not permute:
                    plsc.addupdate_scatter(hist, [addr], ones, mask=m)
                else:
                    pos = plsc.load_gather(cnt, [addr], mask=m)
                    plsc.store_scatter(cnt, [addr], pos + 1, mask=m)
                    if p < 3:
                        tpos = ((pos & (CHUNK - 1)) * LANES
                                + lax.shift_right_logical(pos, 12))
                        plsc.store_scatter(dst, [tpos], iv, mask=m)
                    else:
                        m2 = m & (pos < NOUT)
                        plsc.store_scatter(dst, [pos], iv, mask=m2)
                        plsc.store_scatter(dst, [KEYBASE + pos], kv, mask=m2)

            def hist_win(w, _):
                if p > 0:
                    pltpu.sync_copy(sphbm.at[b].at[pl.ds(w * 2048, 2048)],
                                    win_i)
                lax.fori_loop(0, WIN, lambda t, _: (sweep(w, t, False), 0)[1],
                              0)
                return 0
            lax.fori_loop(0, NWIN, hist_win, 0)

            def scan(d, carry):
                row = hist[pl.ds(d * 16, 16)]
                inc = plsc.cumsum(row)
                cnt[pl.ds(d * 16, 16)] = inc - row + carry
                return carry + jnp.sum(row)
            lax.fori_loop(0, RADIX, scan, jnp.int32(0))

            def perm_win(w, _):
                if p > 0:
                    pltpu.sync_copy(sphbm.at[b].at[pl.ds(w * 2048, 2048)],
                                    win_i)
                lax.fori_loop(0, WIN, lambda t, _: (sweep(w, t, True), 0)[1],
                              0)
                return 0
            lax.fori_loop(0, NWIN, perm_win, 0)

            if p < 3:
                pltpu.sync_copy(dst, sphbm.at[b])

        pltpu.sync_copy(dst.at[pl.ds(0, NOUT)], oidx_hbm.at[b])
        pltpu.sync_copy(dst.at[pl.ds(KEYBASE, NOUT)], okey_hbm.at[b])


def _sc_sort(scores_pad):
    mesh = plsc.VectorSubcoreMesh(core_axis_name="c", subcore_axis_name="s")
    f = pl.kernel(
        _sort_body,
        mesh=mesh,
        compiler_params=pltpu.CompilerParams(needs_layout_passes=False),
        out_type=[
            jax.ShapeDtypeStruct((8, NOUT), jnp.int32),
            jax.ShapeDtypeStruct((8, NOUT), jnp.int32),
            jax.ShapeDtypeStruct((8, NPAD), jnp.int32),
        ],
        scratch_types=[
            pltpu.VMEM((2048,), jnp.float32),
            pltpu.VMEM((2048,), jnp.int32),
            pltpu.VMEM((NREAL + 1200,), jnp.int32),
            pltpu.VMEM((RADIX * LANES,), jnp.int32),
            pltpu.VMEM((RADIX * LANES,), jnp.int32),
            pltpu.VMEM((NPAD,), jnp.int32),
        ],
    )
    return f(scores_pad)


def kernel(h, W, b):
    B, N, D = h.shape
    n_keep = max(int(N * K_RATIO_), 1)
    s = jax.nn.sigmoid(_logits(h, W) + b)
    s_pad = jnp.pad(s, ((0, 0), (0, 51200 - N)))
    idx_s, key_s, _scr = _sc_sort(s_pad)
    idx = idx_s[:, :n_keep]
    key = key_s[:, :n_keep]
    sg = lax.bitcast_convert_type(key ^ -1, jnp.float32)
    idx_full = jnp.broadcast_to(idx[:, :, None], (B, n_keep, D))
    hg = jnp.take_along_axis(h, idx_full, axis=1)
    return hg * sg[:, :, None]


# speedup vs baseline: 2.4687x; 2.4687x over previous
import jax
import jax.numpy as jnp
from jax import lax
from jax.experimental import pallas as pl
from jax.experimental.pallas import tpu as pltpu
from jax.experimental.pallas import tpu_sc as plsc

K_RATIO_ = 0.5
BLK = 2000
LANES = 16
CHUNK = 4096
NPAD = LANES * CHUNK
NREAL = 50000
WIN = 128
NWIN = CHUNK // WIN
RADIX = 256
NOUT = 25600
KEYBASE = 32768


def _logits_body(h_ref, w_ref, out_ref):
    z = h_ref[...]
    w = w_ref[...]
    lg = jax.lax.dot_general(
        z.astype(jnp.bfloat16), w.astype(jnp.bfloat16),
        dimension_numbers=(((2,), (1,)), ((), ())),
        preferred_element_type=jnp.float32,
    )
    out_ref[...] = lg[None]


def _logits(h, W):
    B, N, D = h.shape
    grid = (N // BLK,)
    out = pl.pallas_call(
        _logits_body,
        grid=grid,
        in_specs=[
            pl.BlockSpec((B, BLK, D), lambda j: (0, j, 0)),
            pl.BlockSpec((1, D), lambda j: (0, 0)),
        ],
        out_specs=pl.BlockSpec((1, B, BLK, 1), lambda j: (j, 0, 0, 0)),
        out_shape=jax.ShapeDtypeStruct((N // BLK, B, BLK, 1), jnp.float32),
    )(h, W)
    return out[..., 0].transpose(1, 0, 2).reshape(B, N)


def _sort_body(scores_hbm, oidx_hbm, okey_hbm, sphbm,
               win_f, win_i, ukT, hist, cnt, dst):
    c = lax.axis_index("c")
    s = lax.axis_index("s")
    b = s * 2 + c
    lidx = lax.iota(jnp.int32, LANES)
    zeros16 = jnp.zeros((LANES,), jnp.int32)
    ones = jnp.ones((LANES,), jnp.int32)

    @pl.when(b < 8)
    def _run():
        def ldw(w, _):
            pltpu.sync_copy(scores_hbm.at[b].at[pl.ds(w * 2048, 2048)], win_f)

            def xf(j, _):
                x = win_f[pl.ds(j * 16, 16)]
                ukT[pl.ds(w * 2048 + j * 16, 16)] = (
                    lax.bitcast_convert_type(x, jnp.int32) ^ -1)
                return 0
            lax.fori_loop(0, 128, xf, 0)
            return 0
        lax.fori_loop(0, NREAL // 2048 + 1, ldw, 0)

        for p in range(4):
            shift = 8 * p

            def zero(d, _):
                hist[pl.ds(d * 16, 16)] = zeros16
                return 0
            lax.fori_loop(0, RADIX * LANES // 16, zero, 0)

            def sweep(w, t, permute):
                o = w * WIN + t
                if p == 0:
                    iv = lidx * CHUNK + o
                else:
                    iv = win_i[pl.ds(t * 16, 16)]
                m = (lidx * CHUNK + o) < NREAL
                kv = plsc.load_gather(ukT, [iv], mask=m)
                d = lax.shift_right_logical(kv, shift) & 255
                addr = d * LANES + lidx
                if not permute:
                    plsc.addupdate_scatter(hist, [addr], ones, mask=m)
                else:
                    pos = plsc.load_gather(cnt, [addr], mask=m)
                    plsc.store_scatter(cnt, [addr], pos + 1, mask=m)
                    if p < 3:
                        tpos = ((pos & (CHUNK - 1)) * LANES
                                + lax.shift_right_logical(pos, 12))
                        plsc.store_scatter(dst, [tpos], iv, mask=m)
                    else:
                        m2 = m & (pos < NOUT)
                        plsc.store_scatter(dst, [pos], iv, mask=m2)
                        plsc.store_scatter(dst, [KEYBASE + pos], kv, mask=m2)

            def hist_win(w, _):
                if p > 0:
                    pltpu.sync_copy(sphbm.at[b].at[pl.ds(w * 2048, 2048)],
                                    win_i)
                lax.fori_loop(0, WIN, lambda t, _: (sweep(w, t, False), 0)[1],
                              0)
                return 0
            lax.fori_loop(0, NWIN, hist_win, 0)

            def scan(d, carry):
                row = hist[pl.ds(d * 16, 16)]
                inc = plsc.cumsum(row)
                cnt[pl.ds(d * 16, 16)] = inc - row + carry
                return carry + jnp.sum(row)
            lax.fori_loop(0, RADIX, scan, jnp.int32(0))

            def perm_win(w, _):
                if p > 0:
                    pltpu.sync_copy(sphbm.at[b].at[pl.ds(w * 2048, 2048)],
                                    win_i)
                lax.fori_loop(0, WIN, lambda t, _: (sweep(w, t, True), 0)[1],
                              0)
                return 0
            lax.fori_loop(0, NWIN, perm_win, 0)

            if p < 3:
                pltpu.sync_copy(dst, sphbm.at[b])

        pltpu.sync_copy(dst.at[pl.ds(0, NOUT)], oidx_hbm.at[b])
        pltpu.sync_copy(dst.at[pl.ds(KEYBASE, NOUT)], okey_hbm.at[b])


def _sc_sort(scores_pad):
    mesh = plsc.VectorSubcoreMesh(core_axis_name="c", subcore_axis_name="s")
    f = pl.kernel(
        _sort_body,
        mesh=mesh,
        compiler_params=pltpu.CompilerParams(needs_layout_passes=False),
        out_type=[
            jax.ShapeDtypeStruct((8, NOUT), jnp.int32),
            jax.ShapeDtypeStruct((8, NOUT), jnp.int32),
            jax.ShapeDtypeStruct((8, NPAD), jnp.int32),
        ],
        scratch_types=[
            pltpu.VMEM((2048,), jnp.float32),
            pltpu.VMEM((2048,), jnp.int32),
            pltpu.VMEM((NREAL + 1200,), jnp.int32),
            pltpu.VMEM((RADIX * LANES,), jnp.int32),
            pltpu.VMEM((RADIX * LANES,), jnp.int32),
            pltpu.VMEM((NPAD,), jnp.int32),
        ],
    )
    return f(scores_pad)


NKEEP = 25000
GW = 128
NGW = 196
TAIL = NKEEP - 195 * GW


def _gather_body(h_hbm, idx_hbm, key_hbm, out_hbm, idxv, keyv, rows, sem):
    c = lax.axis_index("c")
    s = lax.axis_index("s")
    wid = s * 2 + c
    b = lax.shift_right_logical(wid, 2)
    q = wid & 3

    def scale(nrows):
        def grp(g, _):
            kv = keyv[pl.ds(g * 16, 16)]
            sv = lax.bitcast_convert_type(kv ^ -1, jnp.float32)
            for r in range(16):
                srow = jnp.broadcast_to(
                    lax.squeeze(lax.slice(sv, (r,), (r + 1,)), (0,)), (16,))
                j = g * 16 + r
                for cc in range(8):
                    x = rows[j, pl.ds(cc * 16, 16)]
                    rows[j, pl.ds(cc * 16, 16)] = x * srow
            return 0
        lax.fori_loop(0, (nrows + 15) // 16, grp, 0)

    def win(i, _):
        w = q + 4 * i
        base = pl.multiple_of(w * GW, GW)
        pltpu.sync_copy(idx_hbm.at[b].at[pl.ds(base, GW)], idxv)
        pltpu.sync_copy(key_hbm.at[b].at[pl.ds(base, GW)], keyv)
        pltpu.async_copy(h_hbm.at[b].at[idxv], rows, sem).wait()
        scale(GW)

        @pl.when(w < NGW - 1)
        def _full():
            pltpu.sync_copy(rows, out_hbm.at[b].at[pl.ds(base, GW)])

        @pl.when(w == NGW - 1)
        def _tail():
            pltpu.sync_copy(rows.at[pl.ds(0, TAIL)],
                            out_hbm.at[b].at[pl.ds(base, TAIL)])
        return 0
    lax.fori_loop(0, (NGW + 3) // 4, win, 0)


def _sc_gather(h, idx_s, key_s):
    B, N, D = h.shape
    mesh = plsc.VectorSubcoreMesh(core_axis_name="c", subcore_axis_name="s")
    f = pl.kernel(
        _gather_body,
        mesh=mesh,
        compiler_params=pltpu.CompilerParams(needs_layout_passes=False),
        out_type=jax.ShapeDtypeStruct((B, NKEEP, D), jnp.float32),
        scratch_types=[
            pltpu.VMEM((GW,), jnp.int32),
            pltpu.VMEM((GW,), jnp.int32),
            pltpu.VMEM((GW, 128), jnp.float32),
            pltpu.SemaphoreType.DMA,
        ],
    )
    return f(h, idx_s, key_s)


def kernel(h, W, b):
    B, N, D = h.shape
    n_keep = max(int(N * K_RATIO_), 1)
    s = jax.nn.sigmoid(_logits(h, W) + b)
    s_pad = jnp.pad(s, ((0, 0), (0, 51200 - N)))
    idx_s, key_s, _scr = _sc_sort(s_pad)
    return _sc_gather(h, idx_s, key_s)


# speedup vs baseline: 2.5348x; 1.0268x over previous
import jax
import jax.numpy as jnp
from jax import lax
from jax.experimental import pallas as pl
from jax.experimental.pallas import tpu as pltpu
from jax.experimental.pallas import tpu_sc as plsc

K_RATIO_ = 0.5
BLK = 2000
LANES = 16
CHUNK = 4096
NPAD = LANES * CHUNK
NREAL = 50000
WIN = 128
NWIN = CHUNK // WIN
RADIX = 256
NOUT = 25600
KEYBASE = 32768


def _logits_body(h_ref, w_ref, out_ref):
    z = h_ref[...]
    w = w_ref[...]
    lg = jax.lax.dot_general(
        z.astype(jnp.bfloat16), w.astype(jnp.bfloat16),
        dimension_numbers=(((2,), (1,)), ((), ())),
        preferred_element_type=jnp.float32,
    )
    out_ref[...] = lg[None]


def _logits(h, W):
    B, N, D = h.shape
    grid = (N // BLK,)
    out = pl.pallas_call(
        _logits_body,
        grid=grid,
        in_specs=[
            pl.BlockSpec((B, BLK, D), lambda j: (0, j, 0)),
            pl.BlockSpec((1, D), lambda j: (0, 0)),
        ],
        out_specs=pl.BlockSpec((1, B, BLK, 1), lambda j: (j, 0, 0, 0)),
        out_shape=jax.ShapeDtypeStruct((N // BLK, B, BLK, 1), jnp.float32),
    )(h, W)
    return out[..., 0].transpose(1, 0, 2).reshape(B, N)


def _sort_body(scores_hbm, oidx_hbm, okey_hbm, sphbm,
               win_f, win_i, ukT, hist, cnt, dst):
    c = lax.axis_index("c")
    s = lax.axis_index("s")
    b = s * 2 + c
    lidx = lax.iota(jnp.int32, LANES)
    zeros16 = jnp.zeros((LANES,), jnp.int32)
    ones = jnp.ones((LANES,), jnp.int32)

    @pl.when(b < 8)
    def _run():
        def ldw(w, _):
            pltpu.sync_copy(scores_hbm.at[b].at[pl.ds(w * 2048, 2048)], win_f)

            def xf(j, _):
                for j2 in range(8):
                    x = win_f[pl.ds((j * 8 + j2) * 16, 16)]
                    ukT[pl.ds(w * 2048 + (j * 8 + j2) * 16, 16)] = (
                        lax.bitcast_convert_type(x, jnp.int32) ^ -1)
                return 0
            lax.fori_loop(0, 16, xf, 0)
            return 0
        lax.fori_loop(0, NREAL // 2048 + 1, ldw, 0)

        for p in range(4):
            shift = 8 * p

            def zero(d, _):
                hist[pl.ds(d * 16, 16)] = zeros16
                return 0
            lax.fori_loop(0, RADIX * LANES // 16, zero, 0)

            def sweep(w, t, permute):
                o = w * WIN + t
                if p == 0:
                    iv = lidx * CHUNK + o
                else:
                    iv = win_i[pl.ds(t * 16, 16)]
                m = (lidx * CHUNK + o) < NREAL
                kv = plsc.load_gather(ukT, [iv], mask=m)
                d = lax.shift_right_logical(kv, shift) & 255
                addr = d * LANES + lidx
                if not permute:
                    plsc.addupdate_scatter(hist, [addr], ones, mask=m)
                else:
                    pos = plsc.load_gather(cnt, [addr], mask=m)
                    plsc.store_scatter(cnt, [addr], pos + 1, mask=m)
                    if p < 3:
                        tpos = ((pos & (CHUNK - 1)) * LANES
                                + lax.shift_right_logical(pos, 12))
                        plsc.store_scatter(dst, [tpos], iv, mask=m)
                    else:
                        m2 = m & (pos < NOUT)
                        plsc.store_scatter(dst, [pos], iv, mask=m2)
                        plsc.store_scatter(dst, [KEYBASE + pos], kv, mask=m2)

            def hist_win(w, _):
                if p > 0:
                    pltpu.sync_copy(sphbm.at[b].at[pl.ds(w * 2048, 2048)],
                                    win_i)

                def hgrp(g, _):
                    for t2 in range(16):
                        sweep(w, g * 16 + t2, False)
                    return 0
                lax.fori_loop(0, WIN // 16, hgrp, 0)
                return 0
            lax.fori_loop(0, NWIN, hist_win, 0)

            def scan(d, carry):
                row = hist[pl.ds(d * 16, 16)]
                inc = plsc.cumsum(row)
                cnt[pl.ds(d * 16, 16)] = inc - row + carry
                return carry + jnp.sum(row)
            lax.fori_loop(0, RADIX, scan, jnp.int32(0))

            def perm_win(w, _):
                if p > 0:
                    pltpu.sync_copy(sphbm.at[b].at[pl.ds(w * 2048, 2048)],
                                    win_i)

                def pgrp(g, _):
                    for t2 in range(16):
                        sweep(w, g * 16 + t2, True)
                    return 0
                lax.fori_loop(0, WIN // 16, pgrp, 0)
                return 0
            lax.fori_loop(0, NWIN, perm_win, 0)

            if p < 3:
                pltpu.sync_copy(dst, sphbm.at[b])

        pltpu.sync_copy(dst.at[pl.ds(0, NOUT)], oidx_hbm.at[b])
        pltpu.sync_copy(dst.at[pl.ds(KEYBASE, NOUT)], okey_hbm.at[b])


def _sc_sort(scores_pad):
    mesh = plsc.VectorSubcoreMesh(core_axis_name="c", subcore_axis_name="s")
    f = pl.kernel(
        _sort_body,
        mesh=mesh,
        compiler_params=pltpu.CompilerParams(needs_layout_passes=False),
        out_type=[
            jax.ShapeDtypeStruct((8, NOUT), jnp.int32),
            jax.ShapeDtypeStruct((8, NOUT), jnp.int32),
            jax.ShapeDtypeStruct((8, NPAD), jnp.int32),
        ],
        scratch_types=[
            pltpu.VMEM((2048,), jnp.float32),
            pltpu.VMEM((2048,), jnp.int32),
            pltpu.VMEM((NREAL + 1200,), jnp.int32),
            pltpu.VMEM((RADIX * LANES,), jnp.int32),
            pltpu.VMEM((RADIX * LANES,), jnp.int32),
            pltpu.VMEM((NPAD,), jnp.int32),
        ],
    )
    return f(scores_pad)


NKEEP = 25000
GW = 128
NGW = 196
TAIL = NKEEP - 195 * GW


def _gather_body(h_hbm, idx_hbm, key_hbm, out_hbm, idxv, keyv, rows, sem):
    c = lax.axis_index("c")
    s = lax.axis_index("s")
    wid = s * 2 + c
    b = lax.shift_right_logical(wid, 2)
    q = wid & 3

    def scale(nrows):
        def grp(g, _):
            kv = keyv[pl.ds(g * 16, 16)]
            sv = lax.bitcast_convert_type(kv ^ -1, jnp.float32)
            for r in range(16):
                srow = jnp.broadcast_to(
                    lax.squeeze(lax.slice(sv, (r,), (r + 1,)), (0,)), (16,))
                j = g * 16 + r
                for cc in range(8):
                    x = rows[j, pl.ds(cc * 16, 16)]
                    rows[j, pl.ds(cc * 16, 16)] = x * srow
            return 0
        lax.fori_loop(0, (nrows + 15) // 16, grp, 0)

    def win(i, _):
        w = q + 4 * i
        base = pl.multiple_of(w * GW, GW)
        pltpu.sync_copy(idx_hbm.at[b].at[pl.ds(base, GW)], idxv)
        pltpu.sync_copy(key_hbm.at[b].at[pl.ds(base, GW)], keyv)
        pltpu.async_copy(h_hbm.at[b].at[idxv], rows, sem).wait()
        scale(GW)

        @pl.when(w < NGW - 1)
        def _full():
            pltpu.sync_copy(rows, out_hbm.at[b].at[pl.ds(base, GW)])

        @pl.when(w == NGW - 1)
        def _tail():
            pltpu.sync_copy(rows.at[pl.ds(0, TAIL)],
                            out_hbm.at[b].at[pl.ds(base, TAIL)])
        return 0
    lax.fori_loop(0, (NGW + 3) // 4, win, 0)


def _sc_gather(h, idx_s, key_s):
    B, N, D = h.shape
    mesh = plsc.VectorSubcoreMesh(core_axis_name="c", subcore_axis_name="s")
    f = pl.kernel(
        _gather_body,
        mesh=mesh,
        compiler_params=pltpu.CompilerParams(needs_layout_passes=False),
        out_type=jax.ShapeDtypeStruct((B, NKEEP, D), jnp.float32),
        scratch_types=[
            pltpu.VMEM((GW,), jnp.int32),
            pltpu.VMEM((GW,), jnp.int32),
            pltpu.VMEM((GW, 128), jnp.float32),
            pltpu.SemaphoreType.DMA,
        ],
    )
    return f(h, idx_s, key_s)


def kernel(h, W, b):
    B, N, D = h.shape
    n_keep = max(int(N * K_RATIO_), 1)
    s = jax.nn.sigmoid(_logits(h, W) + b)
    s_pad = jnp.pad(s, ((0, 0), (0, 51200 - N)))
    idx_s, key_s, _scr = _sc_sort(s_pad)
    return _sc_gather(h, idx_s, key_s)


# speedup vs baseline: 2.9062x; 1.1465x over previous
import jax
import jax.numpy as jnp
from jax import lax
from jax.experimental import pallas as pl
from jax.experimental.pallas import tpu as pltpu
from jax.experimental.pallas import tpu_sc as plsc

K_RATIO_ = 0.5
BLK = 2000
LANES = 16
CHUNK = 4096
NPAD = LANES * CHUNK
NREAL = 50000
WIN = 128
NWIN = CHUNK // WIN
RADIX = 256
NOUT = 25600
KEYBASE = 32768


def _logits_body(h_ref, w_ref, out_ref):
    z = h_ref[...]
    w = w_ref[...]
    lg = jax.lax.dot_general(
        z.astype(jnp.bfloat16), w.astype(jnp.bfloat16),
        dimension_numbers=(((2,), (1,)), ((), ())),
        preferred_element_type=jnp.float32,
    )
    out_ref[...] = lg[None]


def _logits(h, W):
    B, N, D = h.shape
    grid = (N // BLK,)
    out = pl.pallas_call(
        _logits_body,
        grid=grid,
        in_specs=[
            pl.BlockSpec((B, BLK, D), lambda j: (0, j, 0)),
            pl.BlockSpec((1, D), lambda j: (0, 0)),
        ],
        out_specs=pl.BlockSpec((1, B, BLK, 1), lambda j: (j, 0, 0, 0)),
        out_shape=jax.ShapeDtypeStruct((N // BLK, B, BLK, 1), jnp.float32),
    )(h, W)
    return out[..., 0].transpose(1, 0, 2).reshape(B, N)


def _sort_body(scores_hbm, oidx_hbm, okey_hbm, sphbm,
               win_f, win_i, ukT, hist, cnt, dst):
    c = lax.axis_index("c")
    s = lax.axis_index("s")
    b = s * 2 + c
    lidx = lax.iota(jnp.int32, LANES)
    zeros16 = jnp.zeros((LANES,), jnp.int32)
    ones = jnp.ones((LANES,), jnp.int32)

    @pl.when(b < 8)
    def _run():
        def ldw(w, _):
            pltpu.sync_copy(scores_hbm.at[b].at[pl.ds(w * 2048, 2048)], win_f)

            def xf(j, _):
                for j2 in range(8):
                    x = win_f[pl.ds((j * 8 + j2) * 16, 16)]
                    ukT[pl.ds(w * 2048 + (j * 8 + j2) * 16, 16)] = (
                        lax.bitcast_convert_type(x, jnp.int32) ^ -1)
                return 0
            lax.fori_loop(0, 16, xf, 0)
            return 0
        lax.fori_loop(0, NREAL // 2048 + 1, ldw, 0)

        def zero(d, _):
            for d2 in range(8):
                dst[pl.ds((d * 8 + d2) * 16, 16)] = zeros16
            return 0
        lax.fori_loop(0, 4 * RADIX * LANES // 128, zero, 0)

        def hall(g, _):
            for t2 in range(8):
                o = g * 8 + t2
                m = (lidx * CHUNK + o) < NREAL
                kv = plsc.load_gather(ukT, [lidx * CHUNK + o], mask=m)
                for pp in range(4):
                    d = lax.shift_right_logical(kv, 8 * pp) & 255
                    plsc.addupdate_scatter(
                        dst, [pp * 4096 + d * LANES + lidx], ones, mask=m)
            return 0
        lax.fori_loop(0, CHUNK // 8, hall, 0)
        for pp in range(1, 4):
            pltpu.sync_copy(
                dst.at[pl.ds(pp * 4096, 4096)],
                sphbm.at[b].at[pl.ds(NPAD + (pp - 1) * 4096, 4096)])
        def h0cp(d, _):
            for d2 in range(8):
                hist[pl.ds((d * 8 + d2) * 16, 16)] = (
                    dst[pl.ds((d * 8 + d2) * 16, 16)])
            return 0
        lax.fori_loop(0, RADIX * LANES // 128, h0cp, 0)

        for p in range(4):
            shift = 8 * p
            if p > 0:
                pltpu.sync_copy(
                    sphbm.at[b].at[pl.ds(NPAD + (p - 1) * 4096, 4096)], hist)

            def sweep(w, t, permute):
                o = w * WIN + t
                if p == 0:
                    iv = lidx * CHUNK + o
                else:
                    iv = win_i[pl.ds(t * 16, 16)]
                m = (lidx * CHUNK + o) < NREAL
                kv = plsc.load_gather(ukT, [iv], mask=m)
                d = lax.shift_right_logical(kv, shift) & 255
                addr = d * LANES + lidx
                pos = plsc.load_gather(cnt, [addr], mask=m)
                plsc.store_scatter(cnt, [addr], pos + 1, mask=m)
                if p < 3:
                    tpos = ((pos & (CHUNK - 1)) * LANES
                            + lax.shift_right_logical(pos, 12))
                    plsc.store_scatter(dst, [tpos], iv, mask=m)
                else:
                    m2 = m & (pos < NOUT)
                    plsc.store_scatter(dst, [pos], iv, mask=m2)
                    plsc.store_scatter(dst, [KEYBASE + pos], kv, mask=m2)

            def scan(d, carry):
                row = hist[pl.ds(d * 16, 16)]
                inc = plsc.cumsum(row)
                cnt[pl.ds(d * 16, 16)] = inc - row + carry
                return carry + jnp.sum(row)
            lax.fori_loop(0, RADIX, scan, jnp.int32(0))

            def perm_win(w, _):
                if p > 0:
                    pltpu.sync_copy(sphbm.at[b].at[pl.ds(w * 2048, 2048)],
                                    win_i)

                def pgrp(g, _):
                    for t2 in range(16):
                        sweep(w, g * 16 + t2, True)
                    return 0
                lax.fori_loop(0, WIN // 16, pgrp, 0)
                return 0
            lax.fori_loop(0, NWIN, perm_win, 0)

            if p < 3:
                pltpu.sync_copy(dst, sphbm.at[b].at[pl.ds(0, NPAD)])

        pltpu.sync_copy(dst.at[pl.ds(0, NOUT)], oidx_hbm.at[b])
        pltpu.sync_copy(dst.at[pl.ds(KEYBASE, NOUT)], okey_hbm.at[b])


def _sc_sort(scores_pad):
    mesh = plsc.VectorSubcoreMesh(core_axis_name="c", subcore_axis_name="s")
    f = pl.kernel(
        _sort_body,
        mesh=mesh,
        compiler_params=pltpu.CompilerParams(needs_layout_passes=False),
        out_type=[
            jax.ShapeDtypeStruct((8, NOUT), jnp.int32),
            jax.ShapeDtypeStruct((8, NOUT), jnp.int32),
            jax.ShapeDtypeStruct((8, NPAD + 3 * 4096), jnp.int32),
        ],
        scratch_types=[
            pltpu.VMEM((2048,), jnp.float32),
            pltpu.VMEM((2048,), jnp.int32),
            pltpu.VMEM((NREAL + 1200,), jnp.int32),
            pltpu.VMEM((RADIX * LANES,), jnp.int32),
            pltpu.VMEM((RADIX * LANES,), jnp.int32),
            pltpu.VMEM((NPAD,), jnp.int32),
        ],
    )
    return f(scores_pad)


NKEEP = 25000
GW = 128
NGW = 196
TAIL = NKEEP - 195 * GW


def _gather_body(h_hbm, idx_hbm, key_hbm, out_hbm, idxv, keyv, rows, sem):
    c = lax.axis_index("c")
    s = lax.axis_index("s")
    wid = s * 2 + c
    b = lax.shift_right_logical(wid, 2)
    q = wid & 3

    def scale(nrows):
        def grp(g, _):
            kv = keyv[pl.ds(g * 16, 16)]
            sv = lax.bitcast_convert_type(kv ^ -1, jnp.float32)
            for r in range(16):
                srow = jnp.broadcast_to(
                    lax.squeeze(lax.slice(sv, (r,), (r + 1,)), (0,)), (16,))
                j = g * 16 + r
                for cc in range(8):
                    x = rows[j, pl.ds(cc * 16, 16)]
                    rows[j, pl.ds(cc * 16, 16)] = x * srow
            return 0
        lax.fori_loop(0, (nrows + 15) // 16, grp, 0)

    def win(i, _):
        w = q + 4 * i
        base = pl.multiple_of(w * GW, GW)
        pltpu.sync_copy(idx_hbm.at[b].at[pl.ds(base, GW)], idxv)
        pltpu.sync_copy(key_hbm.at[b].at[pl.ds(base, GW)], keyv)
        pltpu.async_copy(h_hbm.at[b].at[idxv], rows, sem).wait()
        scale(GW)

        @pl.when(w < NGW - 1)
        def _full():
            pltpu.sync_copy(rows, out_hbm.at[b].at[pl.ds(base, GW)])

        @pl.when(w == NGW - 1)
        def _tail():
            pltpu.sync_copy(rows.at[pl.ds(0, TAIL)],
                            out_hbm.at[b].at[pl.ds(base, TAIL)])
        return 0
    lax.fori_loop(0, (NGW + 3) // 4, win, 0)


def _sc_gather(h, idx_s, key_s):
    B, N, D = h.shape
    mesh = plsc.VectorSubcoreMesh(core_axis_name="c", subcore_axis_name="s")
    f = pl.kernel(
        _gather_body,
        mesh=mesh,
        compiler_params=pltpu.CompilerParams(needs_layout_passes=False),
        out_type=jax.ShapeDtypeStruct((B, NKEEP, D), jnp.float32),
        scratch_types=[
            pltpu.VMEM((GW,), jnp.int32),
            pltpu.VMEM((GW,), jnp.int32),
            pltpu.VMEM((GW, 128), jnp.float32),
            pltpu.SemaphoreType.DMA,
        ],
    )
    return f(h, idx_s, key_s)


def kernel(h, W, b):
    B, N, D = h.shape
    n_keep = max(int(N * K_RATIO_), 1)
    s = jax.nn.sigmoid(_logits(h, W) + b)
    s_pad = jnp.pad(s, ((0, 0), (0, 51200 - N)))
    idx_s, key_s, _scr = _sc_sort(s_pad)
    return _sc_gather(h, idx_s, key_s)


# speedup vs baseline: 3.0881x; 1.0626x over previous
import jax
import jax.numpy as jnp
from jax import lax
from jax.experimental import pallas as pl
from jax.experimental.pallas import tpu as pltpu
from jax.experimental.pallas import tpu_sc as plsc

K_RATIO_ = 0.5
BLK = 2000
LANES = 16
CHUNK = 4096
NPAD = LANES * CHUNK
NREAL = 50000
WIN = 128
NWIN = CHUNK // WIN
RADIX = 256
NOUT = 25600
KEYBASE = 32768


def _logits_body(h_ref, w_ref, out_ref):
    z = h_ref[...]
    w = w_ref[...]
    lg = jax.lax.dot_general(
        z.astype(jnp.bfloat16), w.astype(jnp.bfloat16),
        dimension_numbers=(((2,), (1,)), ((), ())),
        preferred_element_type=jnp.float32,
    )
    out_ref[...] = lg[None]


def _logits(h, W):
    B, N, D = h.shape
    grid = (N // BLK,)
    out = pl.pallas_call(
        _logits_body,
        grid=grid,
        in_specs=[
            pl.BlockSpec((B, BLK, D), lambda j: (0, j, 0)),
            pl.BlockSpec((1, D), lambda j: (0, 0)),
        ],
        out_specs=pl.BlockSpec((1, B, BLK, 1), lambda j: (j, 0, 0, 0)),
        out_shape=jax.ShapeDtypeStruct((N // BLK, B, BLK, 1), jnp.float32),
    )(h, W)
    return out[..., 0].transpose(1, 0, 2).reshape(B, N)


def _sort_body(scores_hbm, oidx_hbm, okey_hbm, sphbm,
               win_f, win_i, win_j, ukT, hist, cnt, dst, wsem):
    c = lax.axis_index("c")
    s = lax.axis_index("s")
    b = s * 2 + c
    lidx = lax.iota(jnp.int32, LANES)
    zeros16 = jnp.zeros((LANES,), jnp.int32)
    ones = jnp.ones((LANES,), jnp.int32)

    @pl.when(b < 8)
    def _run():
        def ldw(w, _):
            pltpu.sync_copy(scores_hbm.at[b].at[pl.ds(w * 2048, 2048)], win_f)

            def xf(j, _):
                for j2 in range(8):
                    x = win_f[pl.ds((j * 8 + j2) * 16, 16)]
                    ukT[pl.ds(w * 2048 + (j * 8 + j2) * 16, 16)] = (
                        lax.bitcast_convert_type(x, jnp.int32) ^ -1)
                return 0
            lax.fori_loop(0, 16, xf, 0)
            return 0
        lax.fori_loop(0, NREAL // 2048 + 1, ldw, 0)

        def zero(d, _):
            for d2 in range(8):
                dst[pl.ds((d * 8 + d2) * 16, 16)] = zeros16
            return 0
        lax.fori_loop(0, 4 * RADIX * LANES // 128, zero, 0)

        def hall(g, _):
            for t2 in range(8):
                o = g * 8 + t2
                m = (lidx * CHUNK + o) < NREAL
                kv = plsc.load_gather(ukT, [lidx * CHUNK + o], mask=m)
                for pp in range(4):
                    d = lax.shift_right_logical(kv, 8 * pp) & 255
                    plsc.addupdate_scatter(
                        dst, [pp * 4096 + d * LANES + lidx], ones, mask=m)
            return 0
        lax.fori_loop(0, CHUNK // 8, hall, 0)
        for pp in range(1, 4):
            pltpu.sync_copy(
                dst.at[pl.ds(pp * 4096, 4096)],
                sphbm.at[b].at[pl.ds(NPAD + (pp - 1) * 4096, 4096)])
        def h0cp(d, _):
            for d2 in range(8):
                hist[pl.ds((d * 8 + d2) * 16, 16)] = (
                    dst[pl.ds((d * 8 + d2) * 16, 16)])
            return 0
        lax.fori_loop(0, RADIX * LANES // 128, h0cp, 0)

        for p in range(4):
            shift = 8 * p
            if p > 0:
                pltpu.sync_copy(
                    sphbm.at[b].at[pl.ds(NPAD + (p - 1) * 4096, 4096)], hist)

            def sweep(w, t, wbuf):
                o = w * WIN + t
                if p == 0:
                    iv = lidx * CHUNK + o
                else:
                    iv = wbuf[pl.ds(t * 16, 16)]
                m = (lidx * CHUNK + o) < NREAL
                kv = plsc.load_gather(ukT, [iv], mask=m)
                d = lax.shift_right_logical(kv, shift) & 255
                addr = d * LANES + lidx
                pos = plsc.load_gather(cnt, [addr], mask=m)
                plsc.store_scatter(cnt, [addr], pos + 1, mask=m)
                if p < 3:
                    tpos = ((pos & (CHUNK - 1)) * LANES
                            + lax.shift_right_logical(pos, 12))
                    plsc.store_scatter(dst, [tpos], iv, mask=m)
                else:
                    m2 = m & (pos < NOUT)
                    plsc.store_scatter(dst, [pos], iv, mask=m2)
                    plsc.store_scatter(dst, [KEYBASE + pos], kv, mask=m2)

            def scan(d, carry):
                row = hist[pl.ds(d * 16, 16)]
                inc = plsc.cumsum(row)
                cnt[pl.ds(d * 16, 16)] = inc - row + carry
                return carry + jnp.sum(row)
            lax.fori_loop(0, RADIX, scan, jnp.int32(0))

            def body(w, wbuf):
                def pgrp(g, _):
                    for t2 in range(16):
                        sweep(w, g * 16 + t2, wbuf)
                    return 0
                lax.fori_loop(0, WIN // 16, pgrp, 0)

            if p == 0:
                lax.fori_loop(0, NWIN, lambda w, _: (body(w, win_i), 0)[1], 0)
            else:
                def wcp(w, buf):
                    return pltpu.async_copy(
                        sphbm.at[b].at[pl.ds(w * 2048, 2048)], buf, wsem)

                def wwait(buf):
                    pltpu.make_async_copy(
                        sphbm.at[b].at[pl.ds(0, 2048)], buf, wsem).wait()

                wcp(0, win_i)
                wcp(1, win_j)
                wwait(win_i)

                def pair(k, _):
                    w0 = k * 2
                    body(w0, win_i)
                    wcp(w0 + 2, win_i)
                    wwait(win_j)
                    body(w0 + 1, win_j)
                    wcp(w0 + 3, win_j)
                    wwait(win_i)
                    return 0
                lax.fori_loop(0, NWIN // 2, pair, 0)
                wwait(win_j)

            if p < 3:
                pltpu.sync_copy(dst, sphbm.at[b].at[pl.ds(0, NPAD)])

        pltpu.sync_copy(dst.at[pl.ds(0, NOUT)], oidx_hbm.at[b])
        pltpu.sync_copy(dst.at[pl.ds(KEYBASE, NOUT)], okey_hbm.at[b])


def _sc_sort(scores_pad):
    mesh = plsc.VectorSubcoreMesh(core_axis_name="c", subcore_axis_name="s")
    f = pl.kernel(
        _sort_body,
        mesh=mesh,
        compiler_params=pltpu.CompilerParams(needs_layout_passes=False),
        out_type=[
            jax.ShapeDtypeStruct((8, NOUT), jnp.int32),
            jax.ShapeDtypeStruct((8, NOUT), jnp.int32),
            jax.ShapeDtypeStruct((8, NPAD + 3 * 4096), jnp.int32),
        ],
        scratch_types=[
            pltpu.VMEM((2048,), jnp.float32),
            pltpu.VMEM((2048,), jnp.int32),
            pltpu.VMEM((2048,), jnp.int32),
            pltpu.VMEM((NREAL + 1200,), jnp.int32),
            pltpu.VMEM((RADIX * LANES,), jnp.int32),
            pltpu.VMEM((RADIX * LANES,), jnp.int32),
            pltpu.VMEM((NPAD,), jnp.int32),
            pltpu.SemaphoreType.DMA,
        ],
    )
    return f(scores_pad)


NKEEP = 25000
GW = 128
NGW = 196
TAIL = NKEEP - 195 * GW


def _gather_body(h_hbm, idx_hbm, key_hbm, out_hbm, idxv, keyv, rows, sem):
    c = lax.axis_index("c")
    s = lax.axis_index("s")
    wid = s * 2 + c
    b = lax.shift_right_logical(wid, 2)
    q = wid & 3

    def scale(nrows):
        def grp(g, _):
            kv = keyv[pl.ds(g * 16, 16)]
            sv = lax.bitcast_convert_type(kv ^ -1, jnp.float32)
            for r in range(16):
                srow = jnp.broadcast_to(
                    lax.squeeze(lax.slice(sv, (r,), (r + 1,)), (0,)), (16,))
                j = g * 16 + r
                for cc in range(8):
                    x = rows[j, pl.ds(cc * 16, 16)]
                    rows[j, pl.ds(cc * 16, 16)] = x * srow
            return 0
        lax.fori_loop(0, (nrows + 15) // 16, grp, 0)

    def win(i, _):
        w = q + 4 * i
        base = pl.multiple_of(w * GW, GW)
        pltpu.sync_copy(idx_hbm.at[b].at[pl.ds(base, GW)], idxv)
        pltpu.sync_copy(key_hbm.at[b].at[pl.ds(base, GW)], keyv)
        pltpu.async_copy(h_hbm.at[b].at[idxv], rows, sem).wait()
        scale(GW)

        @pl.when(w < NGW - 1)
        def _full():
            pltpu.sync_copy(rows, out_hbm.at[b].at[pl.ds(base, GW)])

        @pl.when(w == NGW - 1)
        def _tail():
            pltpu.sync_copy(rows.at[pl.ds(0, TAIL)],
                            out_hbm.at[b].at[pl.ds(base, TAIL)])
        return 0
    lax.fori_loop(0, (NGW + 3) // 4, win, 0)


def _sc_gather(h, idx_s, key_s):
    B, N, D = h.shape
    mesh = plsc.VectorSubcoreMesh(core_axis_name="c", subcore_axis_name="s")
    f = pl.kernel(
        _gather_body,
        mesh=mesh,
        compiler_params=pltpu.CompilerParams(needs_layout_passes=False),
        out_type=jax.ShapeDtypeStruct((B, NKEEP, D), jnp.float32),
        scratch_types=[
            pltpu.VMEM((GW,), jnp.int32),
            pltpu.VMEM((GW,), jnp.int32),
            pltpu.VMEM((GW, 128), jnp.float32),
            pltpu.SemaphoreType.DMA,
        ],
    )
    return f(h, idx_s, key_s)


def kernel(h, W, b):
    B, N, D = h.shape
    n_keep = max(int(N * K_RATIO_), 1)
    s = jax.nn.sigmoid(_logits(h, W) + b)
    s_pad = jnp.pad(s, ((0, 0), (0, 51200 - N)))
    idx_s, key_s, _scr = _sc_sort(s_pad)
    return _sc_gather(h, idx_s, key_s)


# speedup vs baseline: 3.2164x; 1.0415x over previous
import jax
import jax.numpy as jnp
from jax import lax
from jax.experimental import pallas as pl
from jax.experimental.pallas import tpu as pltpu
from jax.experimental.pallas import tpu_sc as plsc

K_RATIO_ = 0.5
BLK = 2000
LANES = 16
CHUNK = 4096
NPAD = LANES * CHUNK
NREAL = 50000
WIN = 128
NWIN = CHUNK // WIN
RADIX = 256
NOUT = 25600
KEYBASE = 32768


def _logits_body(h_ref, w_ref, out_ref):
    z = h_ref[...]
    w = w_ref[...]
    lg = jax.lax.dot_general(
        z.astype(jnp.bfloat16), w.astype(jnp.bfloat16),
        dimension_numbers=(((2,), (1,)), ((), ())),
        preferred_element_type=jnp.float32,
    )
    out_ref[...] = lg[None]


def _logits(h, W):
    B, N, D = h.shape
    grid = (N // BLK,)
    out = pl.pallas_call(
        _logits_body,
        grid=grid,
        in_specs=[
            pl.BlockSpec((B, BLK, D), lambda j: (0, j, 0)),
            pl.BlockSpec((1, D), lambda j: (0, 0)),
        ],
        out_specs=pl.BlockSpec((1, B, BLK, 1), lambda j: (j, 0, 0, 0)),
        out_shape=jax.ShapeDtypeStruct((N // BLK, B, BLK, 1), jnp.float32),
    )(h, W)
    return out[..., 0].transpose(1, 0, 2).reshape(B, N)


def _sort_body(scores_hbm, oidx_hbm, okey_hbm, sphbm,
               win_f, win_i, win_j, ukT, hist, cnt, dst, wsem):
    c = lax.axis_index("c")
    s = lax.axis_index("s")
    b = s * 2 + c
    lidx = lax.iota(jnp.int32, LANES)
    zeros16 = jnp.zeros((LANES,), jnp.int32)
    ones = jnp.ones((LANES,), jnp.int32)

    @pl.when(b < 8)
    def _run():
        def ldw(w, _):
            pltpu.sync_copy(scores_hbm.at[b].at[pl.ds(w * 2048, 2048)], win_f)

            def xf(j, _):
                for j2 in range(8):
                    x = win_f[pl.ds((j * 8 + j2) * 16, 16)]
                    ukT[pl.ds(w * 2048 + (j * 8 + j2) * 16, 16)] = (
                        lax.bitcast_convert_type(x, jnp.int32) ^ -1)
                return 0
            lax.fori_loop(0, 16, xf, 0)
            return 0
        lax.fori_loop(0, NREAL // 2048 + 1, ldw, 0)

        def zero(d, _):
            for d2 in range(8):
                dst[pl.ds((d * 8 + d2) * 16, 16)] = zeros16
            return 0
        lax.fori_loop(0, 4 * RADIX * LANES // 128, zero, 0)

        def hall(g, _):
            for t2 in range(8):
                o = g * 8 + t2
                m = (lidx * CHUNK + o) < NREAL
                kv = plsc.load_gather(ukT, [lidx * CHUNK + o], mask=m)
                for pp in range(4):
                    d = lax.shift_right_logical(kv, 8 * pp) & 255
                    plsc.addupdate_scatter(
                        dst, [pp * 4096 + d * LANES + lidx], ones, mask=m)
            return 0
        lax.fori_loop(0, CHUNK // 8, hall, 0)
        for pp in range(1, 4):
            pltpu.sync_copy(
                dst.at[pl.ds(pp * 4096, 4096)],
                sphbm.at[b].at[pl.ds(NPAD + (pp - 1) * 4096, 4096)])
        def h0cp(d, _):
            for d2 in range(8):
                hist[pl.ds((d * 8 + d2) * 16, 16)] = (
                    dst[pl.ds((d * 8 + d2) * 16, 16)])
            return 0
        lax.fori_loop(0, RADIX * LANES // 128, h0cp, 0)

        for p in range(4):
            shift = 8 * p
            if p > 0:
                pltpu.sync_copy(
                    sphbm.at[b].at[pl.ds(NPAD + (p - 1) * 4096, 4096)], hist)

            def sweep(w, t, wbuf):
                o = w * WIN + t
                if p == 0:
                    iv = lidx * CHUNK + o
                else:
                    iv = wbuf[pl.ds(t * 16, 16)]
                m = (lidx * CHUNK + o) < NREAL
                kv = plsc.load_gather(ukT, [iv], mask=m)
                d = lax.shift_right_logical(kv, shift) & 255
                addr = d * LANES + lidx
                pos = plsc.load_gather(cnt, [addr], mask=m)
                plsc.store_scatter(cnt, [addr], pos + 1, mask=m)
                if p < 3:
                    tpos = ((pos & (CHUNK - 1)) * LANES
                            + lax.shift_right_logical(pos, 12))
                    plsc.store_scatter(dst, [tpos], iv, mask=m)
                else:
                    m2 = m & (pos < NOUT)
                    plsc.store_scatter(dst, [pos], iv, mask=m2)
                    plsc.store_scatter(dst, [KEYBASE + pos], kv, mask=m2)

            def scan(d, carry):
                row = hist[pl.ds(d * 16, 16)]
                inc = plsc.cumsum(row)
                cnt[pl.ds(d * 16, 16)] = inc - row + carry
                return carry + jnp.sum(row)
            lax.fori_loop(0, RADIX, scan, jnp.int32(0))

            def body(w, wbuf):
                def pgrp(g, _):
                    for t2 in range(16):
                        sweep(w, g * 16 + t2, wbuf)
                    return 0
                lax.fori_loop(0, WIN // 16, pgrp, 0)

            if p == 0:
                lax.fori_loop(0, NWIN, lambda w, _: (body(w, win_i), 0)[1], 0)
            else:
                def wcp(w, buf):
                    return pltpu.async_copy(
                        sphbm.at[b].at[pl.ds(w * 2048, 2048)], buf, wsem)

                def wwait(buf):
                    pltpu.make_async_copy(
                        sphbm.at[b].at[pl.ds(0, 2048)], buf, wsem).wait()

                wcp(0, win_i)
                wcp(1, win_j)
                wwait(win_i)

                def pair(k, _):
                    w0 = k * 2
                    body(w0, win_i)
                    wcp(w0 + 2, win_i)
                    wwait(win_j)
                    body(w0 + 1, win_j)
                    wcp(w0 + 3, win_j)
                    wwait(win_i)
                    return 0
                lax.fori_loop(0, NWIN // 2, pair, 0)
                wwait(win_j)

            if p < 3:
                pltpu.sync_copy(dst, sphbm.at[b].at[pl.ds(0, NPAD)])

        pltpu.sync_copy(dst.at[pl.ds(0, NOUT)], oidx_hbm.at[b])
        pltpu.sync_copy(dst.at[pl.ds(KEYBASE, NOUT)], okey_hbm.at[b])


def _sc_sort(scores_pad):
    mesh = plsc.VectorSubcoreMesh(core_axis_name="c", subcore_axis_name="s")
    f = pl.kernel(
        _sort_body,
        mesh=mesh,
        compiler_params=pltpu.CompilerParams(needs_layout_passes=False),
        out_type=[
            jax.ShapeDtypeStruct((8, NOUT), jnp.int32),
            jax.ShapeDtypeStruct((8, NOUT), jnp.int32),
            jax.ShapeDtypeStruct((8, NPAD + 3 * 4096), jnp.int32),
        ],
        scratch_types=[
            pltpu.VMEM((2048,), jnp.float32),
            pltpu.VMEM((2048,), jnp.int32),
            pltpu.VMEM((2048,), jnp.int32),
            pltpu.VMEM((NREAL + 1200,), jnp.int32),
            pltpu.VMEM((RADIX * LANES,), jnp.int32),
            pltpu.VMEM((RADIX * LANES,), jnp.int32),
            pltpu.VMEM((NPAD,), jnp.int32),
            pltpu.SemaphoreType.DMA,
        ],
    )
    return f(scores_pad)


NKEEP = 25000
GW = 128
NGW = 196
TAIL = NKEEP - 195 * GW


def _gather_body(h_hbm, idx_hbm, key_hbm, out_hbm,
                 ia, ka, ib, kb, rowsa, rowsb, gsem):
    c = lax.axis_index("c")
    s = lax.axis_index("s")
    wid = s * 2 + c
    b = lax.shift_right_logical(wid, 2)
    q = wid & 3

    def base(j):
        return pl.multiple_of((q + 4 * j) * GW, GW)

    def idxcp(j, iv, kv):
        pltpu.sync_copy(idx_hbm.at[b].at[pl.ds(base(j), GW)], iv)
        pltpu.sync_copy(key_hbm.at[b].at[pl.ds(base(j), GW)], kv)

    def gstart(iv, rows):
        pltpu.async_copy(h_hbm.at[b].at[iv], rows, gsem)

    def gwait(iv, rows):
        pltpu.make_async_copy(h_hbm.at[b].at[iv], rows, gsem).wait()

    def scale_out(j, kv, rows):
        def grp(g, _):
            kvv = kv[pl.ds(g * 16, 16)]
            sv = lax.bitcast_convert_type(kvv ^ -1, jnp.float32)
            for r in range(16):
                srow = jnp.broadcast_to(
                    lax.squeeze(lax.slice(sv, (r,), (r + 1,)), (0,)), (16,))
                jr = g * 16 + r
                for cc in range(8):
                    x = rows[jr, pl.ds(cc * 16, 16)]
                    rows[jr, pl.ds(cc * 16, 16)] = x * srow
            return 0
        lax.fori_loop(0, GW // 16, grp, 0)
        w = q + 4 * j

        @pl.when(w < NGW - 1)
        def _full():
            pltpu.sync_copy(rows, out_hbm.at[b].at[pl.ds(base(j), GW)])

        @pl.when(w == NGW - 1)
        def _tail():
            pltpu.sync_copy(rows.at[pl.ds(0, TAIL)],
                            out_hbm.at[b].at[pl.ds(base(j), TAIL)])

    idxcp(0, ia, ka)
    gstart(ia, rowsa)
    idxcp(1, ib, kb)

    def pair(k, _):
        gstart(ib, rowsb)
        gwait(ia, rowsa)
        scale_out(2 * k, ka, rowsa)
        idxcp(2 * k + 2, ia, ka)
        gstart(ia, rowsa)
        gwait(ib, rowsb)
        scale_out(2 * k + 1, kb, rowsb)
        idxcp(2 * k + 3, ib, kb)
        return 0
    lax.fori_loop(0, 24, pair, 0)
    gwait(ia, rowsa)
    scale_out(48, ka, rowsa)


def _sc_gather(h, idx_s, key_s):
    B, N, D = h.shape
    mesh = plsc.VectorSubcoreMesh(core_axis_name="c", subcore_axis_name="s")
    f = pl.kernel(
        _gather_body,
        mesh=mesh,
        compiler_params=pltpu.CompilerParams(needs_layout_passes=False),
        out_type=jax.ShapeDtypeStruct((B, NKEEP, D), jnp.float32),
        scratch_types=[
            pltpu.VMEM((GW,), jnp.int32),
            pltpu.VMEM((GW,), jnp.int32),
            pltpu.VMEM((GW,), jnp.int32),
            pltpu.VMEM((GW,), jnp.int32),
            pltpu.VMEM((GW, 128), jnp.float32),
            pltpu.VMEM((GW, 128), jnp.float32),
            pltpu.SemaphoreType.DMA,
        ],
    )
    return f(h, idx_s, key_s)


def kernel(h, W, b):
    B, N, D = h.shape
    n_keep = max(int(N * K_RATIO_), 1)
    s = jax.nn.sigmoid(_logits(h, W) + b)
    s_pad = jnp.pad(s, ((0, 0), (0, 51200 - N)))
    idx_s, key_s, _scr = _sc_sort(s_pad)
    return _sc_gather(h, idx_s, key_s)
